# Initial kernel scaffold; baseline (speedup 1.0000x reference)
import jax
import jax.numpy as jnp
from jax.experimental import pallas as pl


def kernel(context_ids, doc_ids, target_noise_ids, word_emb, lecture, O):
    raise NotImplementedError("write your pallas kernel here")


# speedup vs baseline: 1.2304x; 1.2304x over previous
import functools

import jax
import jax.numpy as jnp
from jax import lax
from jax.experimental import pallas as pl
from jax.experimental.pallas import tpu as pltpu
from jax.experimental.pallas import tpu_sc as plsc

VOCAB = 100000
DIM = 128
B = 4096
CTX = 20
NOISE = 10

NUM_CORES = 2
NUM_SUBCORES = 16
NW = NUM_CORES * NUM_SUBCORES
BPW = B // NW
JPW = BPW * NOISE
CHUNK = 128
NCHUNK = JPW // CHUNK


def _tr_body(o_ref, ot_ref):
    ot_ref[...] = o_ref[...].T


@jax.jit
def _transpose_tc(o):
    bt = 512
    grid = pl.cdiv(VOCAB, bt)
    return pl.pallas_call(
        _tr_body,
        grid=(grid,),
        in_specs=[pl.BlockSpec((DIM, bt), lambda i: (0, i))],
        out_specs=pl.BlockSpec((bt, DIM), lambda i: (i, 0)),
        out_shape=jax.ShapeDtypeStruct((VOCAB, DIM), jnp.float32),
    )(o)


def _sc_body(ctxT_hbm, doc_hbm, tn_hbm, wemb_hbm, lect_hbm, ot_hbm, out_hbm,
             ctx_v, doc_v, tn_v, x_v, rows_v, out_v, sem1, sem2):
    cid = lax.axis_index("c")
    sid = lax.axis_index("s")
    wid = sid * NUM_CORES + cid
    base = wid * BPW

    pltpu.sync_copy(ctxT_hbm.at[:, pl.ds(base, BPW)], ctx_v)
    pltpu.sync_copy(doc_hbm.at[pl.ds(base, BPW)], doc_v)
    pltpu.sync_copy(tn_hbm.at[pl.ds(base * NOISE, JPW)], tn_v)

    pltpu.async_copy(lect_hbm.at[doc_v], x_v, sem1).wait()
    descs = []
    for c in range(CTX):
        descs.append(
            pltpu.async_copy(wemb_hbm.at[ctx_v.at[c]], x_v, sem2, add=True))
    for d in descs:
        d.wait()

    iota16 = lax.iota(jnp.int32, 16)
    zero16 = jnp.zeros((16,), jnp.float32)

    def chunk_body(ch, _):
        pltpu.async_copy(
            ot_hbm.at[tn_v.at[pl.ds(ch * CHUNK, CHUNK)]], rows_v, sem1
        ).wait()

        def gbody(g, _):
            out16 = zero16
            for m in range(16):
                jj = g * 16 + m
                b = (ch * CHUNK + jj) // NOISE
                acc = x_v[b, pl.ds(0, 16)] * rows_v[jj, pl.ds(0, 16)]
                for k in range(1, DIM // 16):
                    acc = acc + (x_v[b, pl.ds(k * 16, 16)]
                                 * rows_v[jj, pl.ds(k * 16, 16)])
                out16 = jnp.where(iota16 == m, jnp.sum(acc), out16)
            out_v[pl.ds(ch * CHUNK + g * 16, 16)] = out16
            return 0

        lax.fori_loop(0, CHUNK // 16, gbody, 0)
        return 0

    lax.fori_loop(0, NCHUNK, chunk_body, 0)

    pltpu.sync_copy(out_v, out_hbm.at[pl.ds(base * NOISE, JPW)])


@jax.jit
def _sc_call(ctxT, doc_ids, tn_flat, word_emb, lecture, ot):
    mesh = plsc.VectorSubcoreMesh(core_axis_name="c", subcore_axis_name="s")
    f = pl.kernel(
        _sc_body,
        out_type=jax.ShapeDtypeStruct((B * NOISE,), jnp.float32),
        mesh=mesh,
        compiler_params=pltpu.CompilerParams(needs_layout_passes=False),
        scratch_types=[
            pltpu.VMEM((CTX, BPW), jnp.int32),
            pltpu.VMEM((BPW,), jnp.int32),
            pltpu.VMEM((JPW,), jnp.int32),
            pltpu.VMEM((BPW, DIM), jnp.float32),
            pltpu.VMEM((CHUNK, DIM), jnp.float32),
            pltpu.VMEM((JPW,), jnp.float32),
            pltpu.SemaphoreType.DMA,
            pltpu.SemaphoreType.DMA,
        ],
    )
    return f(ctxT, doc_ids, tn_flat, word_emb, lecture, ot)


def kernel(context_ids, doc_ids, target_noise_ids, word_emb, lecture, O):
    ot = _transpose_tc(O)
    ctxT = context_ids.T
    tn_flat = target_noise_ids.reshape(-1)
    out_flat = _sc_call(ctxT, doc_ids, tn_flat, word_emb, lecture, ot)
    return out_flat.reshape(B, NOISE)


# speedup vs baseline: 1.6936x; 1.3765x over previous
import functools

import jax
import jax.numpy as jnp
from jax import lax
from jax.experimental import pallas as pl
from jax.experimental.pallas import tpu as pltpu
from jax.experimental.pallas import tpu_sc as plsc

VOCAB = 100000
DIM = 128
B = 4096
CTX = 20
NOISE = 10

NUM_CORES = 2
NUM_SUBCORES = 16
NW = NUM_CORES * NUM_SUBCORES
BPW = B // NW
JPW = BPW * NOISE
CHUNK = 128
NCHUNK = JPW // CHUNK


def _tr_body(o_ref, ot_ref):
    ot_ref[...] = o_ref[...].T


@jax.jit
def _transpose_tc(o):
    bt = 2048
    grid = pl.cdiv(VOCAB, bt)
    return pl.pallas_call(
        _tr_body,
        grid=(grid,),
        in_specs=[pl.BlockSpec((DIM, bt), lambda i: (0, i))],
        out_specs=pl.BlockSpec((bt, DIM), lambda i: (i, 0)),
        out_shape=jax.ShapeDtypeStruct((VOCAB, DIM), jnp.float32),
    )(o)


def _sc_body(ctxT_hbm, doc_hbm, tn_hbm, wemb_hbm, lect_hbm, ot_hbm, out_hbm,
             ctx_v, doc_v, tn_v, x_v, rows_v, out_v, sem1, sem2):
    cid = lax.axis_index("c")
    sid = lax.axis_index("s")
    wid = sid * NUM_CORES + cid
    base = wid * BPW

    pltpu.sync_copy(ctxT_hbm.at[:, pl.ds(base, BPW)], ctx_v)
    pltpu.sync_copy(doc_hbm.at[pl.ds(base, BPW)], doc_v)
    pltpu.sync_copy(tn_hbm.at[pl.ds(base * NOISE, JPW)], tn_v)

    pltpu.async_copy(lect_hbm.at[doc_v], x_v, sem1).wait()
    descs = []
    for c in range(CTX):
        descs.append(
            pltpu.async_copy(wemb_hbm.at[ctx_v.at[c]], x_v, sem2, add=True))
    for d in descs:
        d.wait()

    iota16 = lax.iota(jnp.int32, 16)
    zero16 = jnp.zeros((16,), jnp.float32)

    def chunk_body(ch, _):
        pltpu.async_copy(
            ot_hbm.at[tn_v.at[pl.ds(ch * CHUNK, CHUNK)]], rows_v, sem1
        ).wait()

        def gbody(g, _):
            out16 = zero16
            for m in range(16):
                jj = g * 16 + m
                b = (ch * CHUNK + jj) // NOISE
                acc = x_v[b, pl.ds(0, 16)] * rows_v[jj, pl.ds(0, 16)]
                for k in range(1, DIM // 16):
                    acc = acc + (x_v[b, pl.ds(k * 16, 16)]
                                 * rows_v[jj, pl.ds(k * 16, 16)])
                out16 = jnp.where(iota16 == m, jnp.sum(acc), out16)
            out_v[pl.ds(ch * CHUNK + g * 16, 16)] = out16
            return 0

        lax.fori_loop(0, CHUNK // 16, gbody, 0)
        return 0

    lax.fori_loop(0, NCHUNK, chunk_body, 0)

    pltpu.sync_copy(out_v, out_hbm.at[pl.ds(base * NOISE, JPW)])


@jax.jit
def _sc_call(ctxT, doc_ids, tn_flat, word_emb, lecture, ot):
    mesh = plsc.VectorSubcoreMesh(core_axis_name="c", subcore_axis_name="s")
    f = pl.kernel(
        _sc_body,
        out_type=jax.ShapeDtypeStruct((B * NOISE,), jnp.float32),
        mesh=mesh,
        compiler_params=pltpu.CompilerParams(needs_layout_passes=False),
        scratch_types=[
            pltpu.VMEM((CTX, BPW), jnp.int32),
            pltpu.VMEM((BPW,), jnp.int32),
            pltpu.VMEM((JPW,), jnp.int32),
            pltpu.VMEM((BPW, DIM), jnp.float32),
            pltpu.VMEM((CHUNK, DIM), jnp.float32),
            pltpu.VMEM((JPW,), jnp.float32),
            pltpu.SemaphoreType.DMA,
            pltpu.SemaphoreType.DMA,
        ],
    )
    return f(ctxT, doc_ids, tn_flat, word_emb, lecture, ot)


def kernel(context_ids, doc_ids, target_noise_ids, word_emb, lecture, O):
    ot = _transpose_tc(O)
    ctxT = context_ids.T
    tn_flat = target_noise_ids.reshape(-1)
    out_flat = _sc_call(ctxT, doc_ids, tn_flat, word_emb, lecture, ot)
    return out_flat.reshape(B, NOISE)


# speedup vs baseline: 1.8693x; 1.1038x over previous
import functools

import jax
import jax.numpy as jnp
from jax import lax
from jax.experimental import pallas as pl
from jax.experimental.pallas import tpu as pltpu
from jax.experimental.pallas import tpu_sc as plsc

VOCAB = 100000
DIM = 128
B = 4096
CTX = 20
NOISE = 10

NUM_CORES = 2
NUM_SUBCORES = 16
NW = NUM_CORES * NUM_SUBCORES
BPW = B // NW
JPW = BPW * NOISE
CHUNK = 128
NCHUNK = JPW // CHUNK

_SC_PARAMS = pltpu.CompilerParams(needs_layout_passes=False)


def _tr_body(o_ref, ot_ref):
    ot_ref[...] = o_ref[...].T


@jax.jit
def _transpose_tc(o):
    bt = 2048
    grid = pl.cdiv(VOCAB, bt)
    return pl.pallas_call(
        _tr_body,
        grid=(grid,),
        in_specs=[pl.BlockSpec((DIM, bt), lambda i: (0, i))],
        out_specs=pl.BlockSpec((bt, DIM), lambda i: (i, 0)),
        out_shape=jax.ShapeDtypeStruct((VOCAB, DIM), jnp.float32),
    )(o)


def _wid_base():
    cid = lax.axis_index("c")
    sid = lax.axis_index("s")
    return (sid * NUM_CORES + cid) * BPW


def _stage1_body(ctxT_hbm, doc_hbm, wemb_hbm, lect_hbm, x_hbm,
                 ctx_v, doc_v, x_v, sem1, sem2):
    base = _wid_base()
    pltpu.sync_copy(ctxT_hbm.at[:, pl.ds(base, BPW)], ctx_v)
    pltpu.sync_copy(doc_hbm.at[pl.ds(base, BPW)], doc_v)

    pltpu.async_copy(lect_hbm.at[doc_v], x_v, sem1).wait()
    descs = []
    for c in range(CTX):
        descs.append(
            pltpu.async_copy(wemb_hbm.at[ctx_v.at[c]], x_v, sem2, add=True))
    for d in descs:
        d.wait()
    pltpu.sync_copy(x_v, x_hbm.at[pl.ds(base, BPW), :])


@jax.jit
def _sc_stage1(ctxT, doc_ids, word_emb, lecture):
    mesh = plsc.VectorSubcoreMesh(core_axis_name="c", subcore_axis_name="s")
    f = pl.kernel(
        _stage1_body,
        out_type=jax.ShapeDtypeStruct((B, DIM), jnp.float32),
        mesh=mesh,
        compiler_params=_SC_PARAMS,
        scratch_types=[
            pltpu.VMEM((CTX, BPW), jnp.int32),
            pltpu.VMEM((BPW,), jnp.int32),
            pltpu.VMEM((BPW, DIM), jnp.float32),
            pltpu.SemaphoreType.DMA,
            pltpu.SemaphoreType.DMA,
        ],
    )
    return f(ctxT, doc_ids, word_emb, lecture)


def _stage2_body(tn_hbm, x_hbm, ot_hbm, out_hbm,
                 tn_v, x_v, rows_a, rows_b, out_v, sem_x, sem_a, sem_b):
    base = _wid_base()
    pltpu.sync_copy(tn_hbm.at[pl.ds(base * NOISE, JPW)], tn_v)
    pltpu.async_copy(ot_hbm.at[tn_v.at[pl.ds(0, CHUNK)]], rows_a, sem_a)
    pltpu.async_copy(x_hbm.at[pl.ds(base, BPW), :], x_v, sem_x).wait()

    iota16 = lax.iota(jnp.int32, 16)
    zero16 = jnp.zeros((16,), jnp.float32)

    def compute(ch, rows_v):
        def gbody(g, _):
            out16 = zero16
            for m in range(16):
                jj = g * 16 + m
                b = (ch * CHUNK + jj) // NOISE
                acc = x_v[b, pl.ds(0, 16)] * rows_v[jj, pl.ds(0, 16)]
                for k in range(1, DIM // 16):
                    acc = acc + (x_v[b, pl.ds(k * 16, 16)]
                                 * rows_v[jj, pl.ds(k * 16, 16)])
                out16 = jnp.where(iota16 == m, jnp.sum(acc), out16)
            out_v[pl.ds(ch * CHUNK + g * 16, 16)] = out16
            return 0

        lax.fori_loop(0, CHUNK // 16, gbody, 0)

    def wait_rows(rows_v, sem):
        pltpu.make_async_copy(ot_hbm.at[pl.ds(0, CHUNK), :], rows_v, sem).wait()

    def pair_body(p, _):
        ch_a = 2 * p
        ch_b = 2 * p + 1
        wait_rows(rows_a, sem_a)
        pltpu.async_copy(
            ot_hbm.at[tn_v.at[pl.ds(ch_b * CHUNK, CHUNK)]], rows_b, sem_b)
        compute(ch_a, rows_a)
        wait_rows(rows_b, sem_b)

        @pl.when(p < NCHUNK // 2 - 1)
        def _():
            pltpu.async_copy(
                ot_hbm.at[tn_v.at[pl.ds((ch_b + 1) * CHUNK, CHUNK)]],
                rows_a, sem_a)

        compute(ch_b, rows_b)
        return 0

    lax.fori_loop(0, NCHUNK // 2, pair_body, 0)
    pltpu.sync_copy(out_v, out_hbm.at[pl.ds(base * NOISE, JPW)])


@jax.jit
def _sc_stage2(tn_flat, x, ot):
    mesh = plsc.VectorSubcoreMesh(core_axis_name="c", subcore_axis_name="s")
    f = pl.kernel(
        _stage2_body,
        out_type=jax.ShapeDtypeStruct((B * NOISE,), jnp.float32),
        mesh=mesh,
        compiler_params=_SC_PARAMS,
        scratch_types=[
            pltpu.VMEM((JPW,), jnp.int32),
            pltpu.VMEM((BPW, DIM), jnp.float32),
            pltpu.VMEM((CHUNK, DIM), jnp.float32),
            pltpu.VMEM((CHUNK, DIM), jnp.float32),
            pltpu.VMEM((JPW,), jnp.float32),
            pltpu.SemaphoreType.DMA,
            pltpu.SemaphoreType.DMA,
            pltpu.SemaphoreType.DMA,
        ],
    )
    return f(tn_flat, x, ot)


def kernel(context_ids, doc_ids, target_noise_ids, word_emb, lecture, O):
    ctxT = context_ids.T
    tn_flat = target_noise_ids.reshape(-1)
    x = _sc_stage1(ctxT, doc_ids, word_emb, lecture)
    ot = _transpose_tc(O)
    out_flat = _sc_stage2(tn_flat, x, ot)
    return out_flat.reshape(B, NOISE)


# speedup vs baseline: 6.3017x; 3.3712x over previous
import jax
import jax.numpy as jnp
from jax import lax
from jax.experimental import pallas as pl
from jax.experimental.pallas import tpu as pltpu
from jax.experimental.pallas import tpu_sc as plsc

VOCAB = 100000
DIM = 128
B = 4096
CTX = 20
NOISE = 10

NUM_CORES = 2
NUM_SUBCORES = 16
NW = NUM_CORES * NUM_SUBCORES
BPW = B // NW
JPW = BPW * NOISE
BCH = 16
CHUNK = BCH * NOISE
NCHUNK = JPW // CHUNK

_SC_PARAMS = pltpu.CompilerParams(needs_layout_passes=False)


def _sc_body(ctxT_hbm, doc_hbm, tn_hbm, wemb_hbm, lect_hbm, ot_hbm, outT_hbm,
             ctx_v, doc_v, tn_v, x_v, rows_a, rows_b, pad_v, outT_v,
             sem1, sem2, sem_a, sem_b):
    cid = lax.axis_index("c")
    sid = lax.axis_index("s")
    base = (sid * NUM_CORES + cid) * BPW

    pltpu.sync_copy(ctxT_hbm.at[:, pl.ds(base, BPW)], ctx_v)
    pltpu.sync_copy(doc_hbm.at[pl.ds(base, BPW)], doc_v)
    pltpu.sync_copy(tn_hbm.at[pl.ds(base * NOISE, JPW)], tn_v)

    pltpu.async_copy(lect_hbm.at[doc_v], x_v, sem1).wait()
    descs = []
    for c in range(CTX):
        descs.append(
            pltpu.async_copy(wemb_hbm.at[ctx_v.at[c]], x_v, sem2, add=True))

    pltpu.async_copy(ot_hbm.at[tn_v.at[pl.ds(0, CHUNK)]], rows_a, sem_a)
    pltpu.async_copy(ot_hbm.at[tn_v.at[pl.ds(CHUNK, CHUNK)]], rows_b, sem_b)

    for d in descs:
        d.wait()

    iota16 = lax.iota(jnp.int32, 16)
    zero16 = jnp.zeros((16,), jnp.float32)
    masks = [iota16 == n for n in range(NOISE)]
    gidx = [iota16 * 16 + n for n in range(NOISE)]

    def compute(ch, rows_v):
        def bbody(bb, _):
            b = ch * BCH + bb
            xr = [x_v[b, pl.ds(k * 16, 16)] for k in range(DIM // 16)]
            out16 = zero16
            for n in range(NOISE):
                jj = bb * NOISE + n
                acc = xr[0] * rows_v[jj, pl.ds(0, 16)]
                for k in range(1, DIM // 16):
                    acc = acc + xr[k] * rows_v[jj, pl.ds(k * 16, 16)]
                out16 = jnp.where(masks[n], jnp.sum(acc), out16)
            pad_v[pl.ds(bb * 16, 16)] = out16
            return 0

        lax.fori_loop(0, BCH, bbody, 0)
        for n in range(NOISE):
            outT_v[n, pl.ds(ch * BCH, BCH)] = plsc.load_gather(pad_v, [gidx[n]])

    def wait_rows(rows_v, sem):
        pltpu.make_async_copy(ot_hbm.at[pl.ds(0, CHUNK), :], rows_v, sem).wait()

    def pair_body(p, _):
        ch_a = 2 * p
        wait_rows(rows_a, sem_a)
        compute(ch_a, rows_a)

        @pl.when(p < NCHUNK // 2 - 1)
        def _():
            pltpu.async_copy(
                ot_hbm.at[tn_v.at[pl.ds((ch_a + 2) * CHUNK, CHUNK)]],
                rows_a, sem_a)

        wait_rows(rows_b, sem_b)
        compute(ch_a + 1, rows_b)

        @pl.when(p < NCHUNK // 2 - 1)
        def _():
            pltpu.async_copy(
                ot_hbm.at[tn_v.at[pl.ds((ch_a + 3) * CHUNK, CHUNK)]],
                rows_b, sem_b)

        return 0

    lax.fori_loop(0, NCHUNK // 2, pair_body, 0)
    pltpu.sync_copy(outT_v, outT_hbm.at[:, pl.ds(base, BPW)])


@jax.jit
def _sc_call(ctxT, doc_ids, tn_flat, word_emb, lecture, ot):
    mesh = plsc.VectorSubcoreMesh(core_axis_name="c", subcore_axis_name="s")
    f = pl.kernel(
        _sc_body,
        out_type=jax.ShapeDtypeStruct((NOISE, B), jnp.float32),
        mesh=mesh,
        compiler_params=_SC_PARAMS,
        scratch_types=[
            pltpu.VMEM((CTX, BPW), jnp.int32),
            pltpu.VMEM((BPW,), jnp.int32),
            pltpu.VMEM((JPW,), jnp.int32),
            pltpu.VMEM((BPW, DIM), jnp.float32),
            pltpu.VMEM((CHUNK, DIM), jnp.float32),
            pltpu.VMEM((CHUNK, DIM), jnp.float32),
            pltpu.VMEM((BCH * 16,), jnp.float32),
            pltpu.VMEM((NOISE, BPW), jnp.float32),
            pltpu.SemaphoreType.DMA,
            pltpu.SemaphoreType.DMA,
            pltpu.SemaphoreType.DMA,
            pltpu.SemaphoreType.DMA,
        ],
    )
    return f(ctxT, doc_ids, tn_flat, word_emb, lecture, ot)


def kernel(context_ids, doc_ids, target_noise_ids, word_emb, lecture, O):
    ctxT = context_ids.T
    tn_flat = target_noise_ids.reshape(-1)
    outT = _sc_call(ctxT, doc_ids, tn_flat, word_emb, lecture, O.T)
    return outT.T


# speedup vs baseline: 6.3658x; 1.0102x over previous
import jax
import jax.numpy as jnp
from jax import lax
from jax.experimental import pallas as pl
from jax.experimental.pallas import tpu as pltpu
from jax.experimental.pallas import tpu_sc as plsc

VOCAB = 100000
DIM = 128
B = 4096
CTX = 20
NOISE = 10

NUM_CORES = 2
NUM_SUBCORES = 16
NW = NUM_CORES * NUM_SUBCORES
BPW = B // NW
JPW = BPW * NOISE
BCH = 16
CHUNK = BCH * NOISE
NCHUNK = JPW // CHUNK

_SC_PARAMS = pltpu.CompilerParams(needs_layout_passes=False)


def _sc_body(ctxT_hbm, doc_hbm, tnT_hbm, wemb_hbm, lect_hbm, ot_hbm, outT_hbm,
             ctx_v, doc_v, tnt_v, tn_v, x_v, rows_a, rows_b, pad_v, outT_v,
             sem1, sem2, sem_a, sem_b):
    cid = lax.axis_index("c")
    sid = lax.axis_index("s")
    base = (sid * NUM_CORES + cid) * BPW

    iota16 = lax.iota(jnp.int32, 16)

    pltpu.sync_copy(ctxT_hbm.at[:, pl.ds(base, BPW)], ctx_v)
    pltpu.sync_copy(doc_hbm.at[pl.ds(base, BPW)], doc_v)
    pltpu.sync_copy(tnT_hbm.at[:, pl.ds(base, BPW)], tnt_v)

    pltpu.async_copy(lect_hbm.at[doc_v], x_v, sem1).wait()
    descs = []
    for c in range(CTX):
        descs.append(
            pltpu.async_copy(wemb_hbm.at[ctx_v.at[c]], x_v, sem2, add=True))

    def tbody(g, _):
        j16 = g * 16 + iota16
        b16 = j16 // NOISE
        n16 = j16 - b16 * NOISE
        tn_v[pl.ds(g * 16, 16)] = plsc.load_gather(tnt_v, [n16, b16])
        return 0

    lax.fori_loop(0, JPW // 16, tbody, 0)

    pltpu.async_copy(ot_hbm.at[tn_v.at[pl.ds(0, CHUNK)]], rows_a, sem_a)
    pltpu.async_copy(ot_hbm.at[tn_v.at[pl.ds(CHUNK, CHUNK)]], rows_b, sem_b)

    for d in descs:
        d.wait()
    zero16 = jnp.zeros((16,), jnp.float32)
    masks = [iota16 == n for n in range(NOISE)]
    gidx = [iota16 * 16 + n for n in range(NOISE)]

    def compute(ch, rows_v):
        def bbody(bb, _):
            b = ch * BCH + bb
            xr = [x_v[b, pl.ds(k * 16, 16)] for k in range(DIM // 16)]
            out16 = zero16
            for n in range(NOISE):
                jj = bb * NOISE + n
                acc = xr[0] * rows_v[jj, pl.ds(0, 16)]
                for k in range(1, DIM // 16):
                    acc = acc + xr[k] * rows_v[jj, pl.ds(k * 16, 16)]
                out16 = jnp.where(masks[n], jnp.sum(acc), out16)
            pad_v[pl.ds(bb * 16, 16)] = out16
            return 0

        lax.fori_loop(0, BCH, bbody, 0)
        for n in range(NOISE):
            outT_v[n, pl.ds(ch * BCH, BCH)] = plsc.load_gather(pad_v, [gidx[n]])

    def wait_rows(rows_v, sem):
        pltpu.make_async_copy(ot_hbm.at[pl.ds(0, CHUNK), :], rows_v, sem).wait()

    def pair_body(p, _):
        ch_a = 2 * p
        wait_rows(rows_a, sem_a)
        compute(ch_a, rows_a)

        @pl.when(p < NCHUNK // 2 - 1)
        def _():
            pltpu.async_copy(
                ot_hbm.at[tn_v.at[pl.ds((ch_a + 2) * CHUNK, CHUNK)]],
                rows_a, sem_a)

        wait_rows(rows_b, sem_b)
        compute(ch_a + 1, rows_b)

        @pl.when(p < NCHUNK // 2 - 1)
        def _():
            pltpu.async_copy(
                ot_hbm.at[tn_v.at[pl.ds((ch_a + 3) * CHUNK, CHUNK)]],
                rows_b, sem_b)

        return 0

    lax.fori_loop(0, NCHUNK // 2, pair_body, 0)
    pltpu.sync_copy(outT_v, outT_hbm.at[:, pl.ds(base, BPW)])


@jax.jit
def _sc_call(ctxT, doc_ids, tnT, word_emb, lecture, ot):
    mesh = plsc.VectorSubcoreMesh(core_axis_name="c", subcore_axis_name="s")
    f = pl.kernel(
        _sc_body,
        out_type=jax.ShapeDtypeStruct((NOISE, B), jnp.float32),
        mesh=mesh,
        compiler_params=_SC_PARAMS,
        scratch_types=[
            pltpu.VMEM((CTX, BPW), jnp.int32),
            pltpu.VMEM((BPW,), jnp.int32),
            pltpu.VMEM((NOISE, BPW), jnp.int32),
            pltpu.VMEM((JPW,), jnp.int32),
            pltpu.VMEM((BPW, DIM), jnp.float32),
            pltpu.VMEM((CHUNK, DIM), jnp.float32),
            pltpu.VMEM((CHUNK, DIM), jnp.float32),
            pltpu.VMEM((BCH * 16,), jnp.float32),
            pltpu.VMEM((NOISE, BPW), jnp.float32),
            pltpu.SemaphoreType.DMA,
            pltpu.SemaphoreType.DMA,
            pltpu.SemaphoreType.DMA,
            pltpu.SemaphoreType.DMA,
        ],
    )
    return f(ctxT, doc_ids, tnT, word_emb, lecture, ot)


def kernel(context_ids, doc_ids, target_noise_ids, word_emb, lecture, O):
    outT = _sc_call(context_ids.T, doc_ids, target_noise_ids.T,
                    word_emb, lecture, O.T)
    return outT.T
